# baseline (speedup 1.0000x reference)
import functools

import jax
import jax.numpy as jnp
import numpy as np
from jax import lax
from jax.experimental import pallas as pl
from jax.experimental.pallas import tpu as pltpu
from jax.experimental.pallas import tpu_sc as plsc

N = 100000
E = 3200000
B = 64
NC, NS, NW = 2, 16, 32
N_PAD = 101376
NPT = N_PAD // NS
NPW = N_PAD // NW
NCHW = NPW // 16
EPW = 100096
E_PAD = EPW * NW
ECH = EPW // 128
HB = B * 256
I32MIN = np.int32(-2147483648)
F1 = np.float32(1.0)
F0 = np.float32(0.0)

MESH = plsc.VectorSubcoreMesh(core_axis_name="c", subcore_axis_name="s")


def _iota16():
  return lax.iota(jnp.int32, 16)


def _mo8(x):
  return pl.multiple_of(x, 8)


def _rsqrt(x):
  i = lax.bitcast_convert_type(x, jnp.int32)
  i = np.int32(0x5F3759DF) - (i >> 1)
  y = lax.bitcast_convert_type(i, jnp.float32)
  for _ in range(3):
    y = y * (np.float32(1.5) - np.float32(0.5) * x * y * y)
  return y


def _tanh(x):
  e = jnp.exp(x + x)
  return F1 - np.float32(2.0) / (e + F1)


def _ceil_pos(t):
  ti = t.astype(jnp.int32).astype(jnp.float32)
  return jnp.where(ti < t, ti + F1, ti)


def _rkey(yv):
  bits = lax.bitcast_convert_type(yv, jnp.int32)
  m = bits >> 31
  return bits ^ (m & np.int32(0x7FFFFFFF)) ^ I32MIN


def _sc32(x):
  return np.int32(x - (1 << 32) if x >= (1 << 31) else x)


def _an_pass(ha, hb, rr):
  def s_body(i, acc):
    v = ha[pl.ds(16 * i, 16)] + hb[pl.ds(16 * i, 16)]
    return acc + jnp.sum(v)
  total = lax.fori_loop(0, 16, s_body, F0)
  thr = total - rr
  def c_body(i, carry):
    run, cnt = carry
    v = ha[pl.ds(16 * i, 16)] + hb[pl.ds(16 * i, 16)]
    cs = plsc.cumsum(v)
    pex = run + cs - v
    cnt = cnt + jnp.sum(jnp.where(pex <= thr, F1, F0))
    return (run + jnp.sum(v), cnt)
  _, cnt = lax.fori_loop(0, 16, c_body, (F0, F0))
  bstar = cnt.astype(jnp.int32) - 1
  def a_body(i, acc):
    v = ha[pl.ds(16 * i, 16)] + hb[pl.ds(16 * i, 16)]
    bidx = 16 * i + _iota16()
    return acc + jnp.sum(jnp.where(bidx <= bstar, v, F0))
  pincl = lax.fori_loop(0, 16, a_body, F0)
  return bstar, rr - (total - pincl)


def _store1(ref, idx, val):
  if getattr(val, "ndim", 0) == 0 or np.ndim(val) == 0:
    val = jnp.full((16,), val)
  plsc.store_scatter(ref, [jnp.full((16,), idx, jnp.int32)],
                     val, mask=_iota16() == 0)


def _sread(ref, idx):
  return plsc.load_gather(ref, [jnp.full((16,), idx, jnp.int32)])[0]


def _add_rows(src_ref, sbase, dst_sh, idxbuf, base, nrow):
  iot = _iota16()
  def body(jj, _):
    for u in range(8):
      idxbuf[pl.ds(16 * u, 16)] = base + 128 * jj + 16 * u + iot
    pltpu.sync_copy(src_ref.at[pl.ds(_mo8(sbase + 128 * jj), 128)],
                    dst_sh.at[idxbuf], add=True)
    return 0
  lax.fori_loop(0, nrow // 128, body, 0)


def _chain(nsteps, hist_h, ha, hb, g, cv):
  c = _sread(cv, g)
  k = _ceil_pos(np.float32(0.8) * c)
  rr = k
  pfx = jnp.int32(0)
  for j in range(nsteps):
    pltpu.sync_copy(hist_h.at[pl.ds(_mo8((j * NC) * HB + g * 256), 256)], ha)
    pltpu.sync_copy(hist_h.at[pl.ds(_mo8((j * NC + 1) * HB + g * 256), 256)], hb)
    bstar, rr = _an_pass(ha, hb, rr)
    pfx = pfx | (bstar << (24 - 8 * j))
  return k, pfx, rr


@functools.partial(
    pl.kernel,
    out_type=(jax.ShapeDtypeStruct((64,), jnp.int32),
              jax.ShapeDtypeStruct((64,), jnp.int32),
              jax.ShapeDtypeStruct((64,), jnp.float32)),
    mesh=MESH,
    compiler_params=pltpu.CompilerParams(needs_layout_passes=False),
    scratch_types=[
        pltpu.VMEM((NPT,), jnp.int32),
        pltpu.VMEM((128,), jnp.float32),
        pltpu.VMEM((64,), jnp.int32),
        pltpu.VMEM((64,), jnp.int32),
        pltpu.VMEM((64,), jnp.float32),
    ])
def _k0_segs(seg_h, st_o, en_o, cn_o, sgw, hist, stv, env, cbv):
  cid = lax.axis_index("c")
  sid = lax.axis_index("s")
  @pl.when((cid == 0) & (sid == 0))
  def _():
    for i in range(8):
      hist[pl.ds(16 * i, 16)] = jnp.zeros((16,), jnp.float32)
    ones = jnp.full((16,), F1)
    def wbody(j, _):
      pltpu.sync_copy(seg_h.at[pl.ds(_mo8(j * NPT), NPT)], sgw)
      def cbody(q, _2):
        gi = j * NPT + 16 * q + _iota16()
        v = sgw[pl.ds(16 * q, 16)]
        plsc.addupdate_scatter(hist, [v], ones, mask=gi < N)
        return 0
      return lax.fori_loop(0, NPT // 16, cbody, 0)
    lax.fori_loop(0, NS, wbody, 0)
    def sbody(m, run):
      v = hist[pl.ds(16 * m, 16)]
      cs = plsc.cumsum(v)
      stv[pl.ds(16 * m, 16)] = (run + cs - v).astype(jnp.int32)
      env[pl.ds(16 * m, 16)] = (run + cs).astype(jnp.int32)
      cbv[pl.ds(16 * m, 16)] = v
      return run + jnp.sum(v)
    lax.fori_loop(0, 4, sbody, F0)
    pltpu.sync_copy(stv, st_o)
    pltpu.sync_copy(env, en_o)
    pltpu.sync_copy(cbv, cn_o)


def _make_deg(masked):
  @functools.partial(
      pl.kernel,
      out_type=(jax.ShapeDtypeStruct((N_PAD,), jnp.float32),
                jax.ShapeDtypeStruct((N_PAD,), jnp.float32)),
      mesh=MESH,
      compiler_params=pltpu.CompilerParams(needs_layout_passes=False),
      scratch_types=[
          pltpu.VMEM((128,), jnp.int32),
          pltpu.VMEM((128,), jnp.int32),
          pltpu.VMEM((128,), jnp.float32),
          pltpu.VMEM((NPT,), jnp.float32),
          pltpu.VMEM_SHARED((N_PAD,), jnp.float32),
          pltpu.SemaphoreType.DMA,
      ])
  def k(row_h, col_h, nmf_h, z1_h, o0, o1, colv, rowv, valv, stage, raw_sh, sem):
    cid = lax.axis_index("c")
    sid = lax.axis_index("s")
    w = cid * NS + sid
    pltpu.sync_copy(z1_h, stage)
    pltpu.sync_copy(stage, raw_sh.at[pl.ds(_mo8(sid * NPT), NPT)])
    if not masked:
      for i in range(8):
        valv[pl.ds(16 * i, 16)] = jnp.full((16,), F1)
    plsc.subcore_barrier()
    base = w * EPW
    def body(i, _):
      off = base + i * 128
      pltpu.sync_copy(col_h.at[pl.ds(_mo8(off), 128)], colv)
      if masked:
        pltpu.sync_copy(row_h.at[pl.ds(_mo8(off), 128)], rowv)
        pltpu.async_copy(nmf_h.at[rowv], valv, sem).wait()
      pltpu.sync_copy(valv, raw_sh.at[colv], add=True)
      return 0
    lax.fori_loop(0, ECH, body, 0)
    plsc.subcore_barrier()
    pltpu.sync_copy(raw_sh.at[pl.ds(_mo8(sid * NPT), NPT)], stage)
    @pl.when(cid == 0)
    def _():
      pltpu.sync_copy(stage, o0.at[pl.ds(_mo8(sid * NPT), NPT)])
    @pl.when(cid == 1)
    def _():
      pltpu.sync_copy(stage, o1.at[pl.ds(_mo8(sid * NPT), NPT)])
  return k

_k1_deg0 = _make_deg(False)
_k1_deg = _make_deg(True)


@functools.partial(
    pl.kernel,
    out_type=(jax.ShapeDtypeStruct((3 * N_PAD,), jnp.float32),
              jax.ShapeDtypeStruct((N_PAD,), jnp.float32)),
    mesh=MESH,
    compiler_params=pltpu.CompilerParams(needs_layout_passes=False),
    scratch_types=[
        pltpu.VMEM((NPW,), jnp.float32),
        pltpu.VMEM((NPW,), jnp.float32),
        pltpu.VMEM((NPW,), jnp.float32),
        pltpu.VMEM((3 * NPW,), jnp.float32),
        pltpu.VMEM((3 * NPW,), jnp.float32),
        pltpu.VMEM((NPW,), jnp.float32),
        pltpu.VMEM((16,), jnp.float32),
    ])
def _k2_nodeA(raw0_h, raw1_h, h_h, nmf_h, wf_h, u_o, dinv_o,
              r0, r1, nmb, hbuf, ubuf, dvb, wv):
  cid = lax.axis_index("c")
  sid = lax.axis_index("s")
  w = cid * NS + sid
  nb = w * NPW
  pltpu.sync_copy(raw0_h.at[pl.ds(_mo8(nb), NPW)], r0)
  pltpu.sync_copy(raw1_h.at[pl.ds(_mo8(nb), NPW)], r1)
  pltpu.sync_copy(nmf_h.at[pl.ds(_mo8(nb), NPW)], nmb)
  for d in range(3):
    pltpu.sync_copy(h_h.at[pl.ds(_mo8(d * N_PAD + nb), NPW)],
                    hbuf.at[pl.ds(d * NPW, NPW)])
  pltpu.sync_copy(wf_h, wv)
  wvec = wv[pl.ds(0, 16)]
  def body(j, _):
    sl = pl.ds(16 * j, 16)
    nmfv = nmb[sl]
    rawv = r0[sl] + r1[sl]
    deg = nmfv * (rawv + F1)
    dv = jnp.where(deg > F0, _rsqrt(deg), F0)
    h0 = hbuf[pl.ds(16 * j, 16)]
    h1 = hbuf[pl.ds(NPW + 16 * j, 16)]
    h2 = hbuf[pl.ds(2 * NPW + 16 * j, 16)]
    for d in range(3):
      xw = h0 * wvec[d] + h1 * wvec[3 + d] + h2 * wvec[6 + d]
      ubuf[pl.ds(d * NPW + 16 * j, 16)] = xw * dv
    dvb[sl] = dv
    return 0
  lax.fori_loop(0, NCHW, body, 0)
  for d in range(3):
    pltpu.sync_copy(ubuf.at[pl.ds(d * NPW, NPW)],
                    u_o.at[pl.ds(_mo8(d * N_PAD + nb), NPW)])
  pltpu.sync_copy(dvb, dinv_o.at[pl.ds(_mo8(nb), NPW)])


@functools.partial(
    pl.kernel,
    out_type=(jax.ShapeDtypeStruct((3 * N_PAD,), jnp.float32),
              jax.ShapeDtypeStruct((3 * N_PAD,), jnp.float32)),
    mesh=MESH,
    compiler_params=pltpu.CompilerParams(needs_layout_passes=False),
    scratch_types=[
        pltpu.VMEM((128,), jnp.int32),
        pltpu.VMEM((128,), jnp.int32),
        pltpu.VMEM((128,), jnp.int32),
        pltpu.VMEM((128,), jnp.int32),
        pltpu.VMEM((128,), jnp.float32),
        pltpu.VMEM((NPT,), jnp.float32),
        pltpu.VMEM_SHARED((3 * N_PAD,), jnp.float32),
        pltpu.SemaphoreType.DMA,
    ])
def _k3_msg(row_h, col_h, u_h, z1_h, o0, o1,
            colv, rowv, gidx, sidx, msgv, stage, s_sh, sem):
  cid = lax.axis_index("c")
  sid = lax.axis_index("s")
  w = cid * NS + sid
  pltpu.sync_copy(z1_h, stage)
  for d in range(3):
    pltpu.sync_copy(stage, s_sh.at[pl.ds(_mo8(d * N_PAD + sid * NPT), NPT)])
  plsc.subcore_barrier()
  base = w * EPW
  def body(i, _):
    off = base + i * 128
    pltpu.sync_copy(col_h.at[pl.ds(_mo8(off), 128)], colv)
    pltpu.sync_copy(row_h.at[pl.ds(_mo8(off), 128)], rowv)
    for d in range(3):
      for u in range(8):
        su = pl.ds(16 * u, 16)
        gidx[su] = rowv[su] + np.int32(d * N_PAD)
        sidx[su] = colv[su] + np.int32(d * N_PAD)
      pltpu.async_copy(u_h.at[gidx], msgv, sem).wait()
      pltpu.sync_copy(msgv, s_sh.at[sidx], add=True)
    return 0
  lax.fori_loop(0, ECH, body, 0)
  plsc.subcore_barrier()
  for d in range(3):
    pltpu.sync_copy(s_sh.at[pl.ds(_mo8(d * N_PAD + sid * NPT), NPT)], stage)
    @pl.when(cid == 0)
    def _():
      pltpu.sync_copy(stage, o0.at[pl.ds(_mo8(d * N_PAD + sid * NPT), NPT)])
    @pl.when(cid == 1)
    def _():
      pltpu.sync_copy(stage, o1.at[pl.ds(_mo8(d * N_PAD + sid * NPT), NPT)])


@functools.partial(
    pl.kernel,
    out_type=(jax.ShapeDtypeStruct((3 * N_PAD,), jnp.float32),
              jax.ShapeDtypeStruct((N_PAD,), jnp.float32)),
    mesh=MESH,
    compiler_params=pltpu.CompilerParams(needs_layout_passes=False),
    scratch_types=[
        pltpu.VMEM((3 * NPW,), jnp.float32),
        pltpu.VMEM((3 * NPW,), jnp.float32),
        pltpu.VMEM((3 * NPW,), jnp.float32),
        pltpu.VMEM((NPW,), jnp.float32),
        pltpu.VMEM((3 * NPW,), jnp.float32),
        pltpu.VMEM((NPW,), jnp.float32),
        pltpu.VMEM((16,), jnp.float32),
        pltpu.VMEM((16,), jnp.float32),
    ])
def _k4_nodeB(s0_h, s1_h, u_h, dinv_h, bf_h, pf_h, h_o, y_o,
              s0b, s1b, ub, dvb, hob, yb, bv, pv):
  cid = lax.axis_index("c")
  sid = lax.axis_index("s")
  w = cid * NS + sid
  nb = w * NPW
  for d in range(3):
    pltpu.sync_copy(s0_h.at[pl.ds(_mo8(d * N_PAD + nb), NPW)],
                    s0b.at[pl.ds(d * NPW, NPW)])
    pltpu.sync_copy(s1_h.at[pl.ds(_mo8(d * N_PAD + nb), NPW)],
                    s1b.at[pl.ds(d * NPW, NPW)])
    pltpu.sync_copy(u_h.at[pl.ds(_mo8(d * N_PAD + nb), NPW)],
                    ub.at[pl.ds(d * NPW, NPW)])
  pltpu.sync_copy(dinv_h.at[pl.ds(_mo8(nb), NPW)], dvb)
  pltpu.sync_copy(bf_h, bv)
  pltpu.sync_copy(pf_h, pv)
  pvec = pv[pl.ds(0, 16)]
  bvec = bv[pl.ds(0, 16)]
  pp = jnp.sum(pvec * pvec)
  rsv = _rsqrt(jnp.full((16,), pp))
  def body(j, _):
    sl = pl.ds(16 * j, 16)
    dv = dvb[sl]
    yv = jnp.zeros((16,), jnp.float32)
    for d in range(3):
      dsl = pl.ds(d * NPW + 16 * j, 16)
      sd = s0b[dsl] + s1b[dsl] + ub[dsl]
      hd = jnp.maximum(dv * sd + bvec[d], F0)
      hob[dsl] = hd
      yv = yv + hd * pvec[d]
    yb[sl] = yv * rsv
    return 0
  lax.fori_loop(0, NCHW, body, 0)
  for d in range(3):
    pltpu.sync_copy(hob.at[pl.ds(d * NPW, NPW)],
                    h_o.at[pl.ds(_mo8(d * N_PAD + nb), NPW)])
  pltpu.sync_copy(yb, y_o.at[pl.ds(_mo8(nb), NPW)])


def _make_hist(r):
  maskdec = _sc32((0xFFFFFFFF << (32 - 8 * r)) & 0xFFFFFFFF) if r else np.int32(0)
  sh = 24 - 8 * r

  @functools.partial(
      pl.kernel,
      out_type=(jax.ShapeDtypeStruct((HB,), jnp.float32),
                jax.ShapeDtypeStruct((HB,), jnp.float32)),
      mesh=MESH,
      compiler_params=pltpu.CompilerParams(needs_layout_passes=False),
      scratch_types=[
          pltpu.VMEM((NPW,), jnp.float32),
          pltpu.VMEM((NPW,), jnp.float32),
          pltpu.VMEM((NPW,), jnp.int32),
          pltpu.VMEM((128,), jnp.float32),
          pltpu.VMEM((256,), jnp.float32),
          pltpu.VMEM((256,), jnp.float32),
          pltpu.VMEM((HB,), jnp.float32),
          pltpu.VMEM((128,), jnp.int32),
          pltpu.VMEM((128,), jnp.int32),
          pltpu.VMEM_SHARED((HB,), jnp.float32),
      ])
  def k(hist_h, y_h, nmf_h, seg_h, cnt_h, z16_h, o0, o1,
        yb, nmb, sgb, cv, ha, hb, hp, pfv, idxw, hs_sh):
    cid = lax.axis_index("c")
    sid = lax.axis_index("s")
    w = cid * NS + sid
    nb = w * NPW
    pltpu.sync_copy(y_h.at[pl.ds(_mo8(nb), NPW)], yb)
    pltpu.sync_copy(nmf_h.at[pl.ds(_mo8(nb), NPW)], nmb)
    pltpu.sync_copy(seg_h.at[pl.ds(_mo8(nb), NPW)], sgb)
    pltpu.sync_copy(cnt_h, cv.at[pl.ds(0, 64)])
    pltpu.sync_copy(z16_h, hp)
    @pl.when(sid == 0)
    def _():
      pltpu.sync_copy(hp, hs_sh)
    plsc.subcore_barrier()
    gmin = sgb[pl.ds(0, 16)][0]
    gmax = sgb[pl.ds(NPW - 16, 16)][15]
    def chain_g(g, _):
      _k, pfx, _rr = _chain(r, hist_h, ha, hb, g, cv)
      _store1(pfv, g, pfx)
      return 0
    lax.fori_loop(gmin, gmax + 1, chain_g, 0)
    ones = jnp.full((16,), F1)
    def hbody(j, _):
      sl = pl.ds(16 * j, 16)
      yv = yb[sl]
      nmv = nmb[sl]
      sgv = sgb[sl]
      rk = _rkey(yv)
      pfn = plsc.load_gather(pfv, [sgv])
      match = (nmv > F0) & ((rk & maskdec) == pfn)
      byte = (rk >> sh) & np.int32(255)
      plsc.addupdate_scatter(hp, [sgv * 256 + byte], ones, mask=match)
      return 0
    lax.fori_loop(0, NCHW, hbody, 0)
    plsc.subcore_barrier()
    _add_rows(hp, 0, hs_sh, idxw, 0, HB)
    plsc.subcore_barrier()
    @pl.when(sid == 0)
    def _():
      pltpu.sync_copy(hs_sh, hp)
      @pl.when(cid == 0)
      def _():
        pltpu.sync_copy(hp, o0)
      @pl.when(cid == 1)
      def _():
        pltpu.sync_copy(hp, o1)
  return k

_k5_hist = [_make_hist(r) for r in range(4)]


@functools.partial(
    pl.kernel,
    out_type=(jax.ShapeDtypeStruct((4 * N_PAD,), jnp.float32),
              jax.ShapeDtypeStruct((4 * N_PAD,), jnp.float32),
              jax.ShapeDtypeStruct((256,), jnp.float32),
              jax.ShapeDtypeStruct((256,), jnp.float32)),
    mesh=MESH,
    compiler_params=pltpu.CompilerParams(needs_layout_passes=False),
    scratch_types=[
        pltpu.VMEM((1024,), jnp.float32),
        pltpu.VMEM((1024,), jnp.float32),
        pltpu.VMEM((3 * 1024,), jnp.float32),
        pltpu.VMEM((4 * 1024,), jnp.float32),
        pltpu.VMEM((256,), jnp.float32),
        pltpu.VMEM((256,), jnp.float32),
        pltpu.VMEM((128,), jnp.int32),
        pltpu.VMEM((128,), jnp.int32),
        pltpu.VMEM((128,), jnp.float32),
        pltpu.VMEM((256,), jnp.float32),
        pltpu.VMEM((NPT,), jnp.float32),
        pltpu.VMEM((128,), jnp.int32),
        pltpu.VMEM_SHARED((4 * N_PAD,), jnp.float32),
        pltpu.VMEM_SHARED((256,), jnp.float32),
    ])
def _k6_keep(hist_h, st_h, en_h, cnt_h, y_h, nmf_h, h_h, z16_h, z1_h,
             ht_o0, ht_o1, gap_o0, gap_o1,
             yw, nmw, hw, outw, ha, hb, stv, env, cv, gapt, stage,
             idxw, s_sh, gap_sh):
  cid = lax.axis_index("c")
  sid = lax.axis_index("s")
  w = cid * NS + sid
  pltpu.sync_copy(z1_h, stage)
  for d in range(4):
    pltpu.sync_copy(stage, s_sh.at[pl.ds(_mo8(d * N_PAD + sid * NPT), NPT)])
  pltpu.sync_copy(z16_h.at[pl.ds(0, 256)], gapt)
  @pl.when(sid == 0)
  def _():
    pltpu.sync_copy(gapt, gap_sh)
  pltpu.sync_copy(st_h, stv.at[pl.ds(0, 64)])
  pltpu.sync_copy(en_h, env.at[pl.ds(0, 64)])
  pltpu.sync_copy(cnt_h, cv.at[pl.ds(0, 64)])
  plsc.subcore_barrier()
  iot = _iota16()
  for q in range(2):
    g = 2 * w + q
    k, pfx, rr = _chain(4, hist_h, ha, hb, g, cv)
    ts = pfx ^ I32MIN
    rri = rr.astype(jnp.int32)
    st = _sread(stv, g)
    en = _sread(env, g)
    ast = st & np.int32(-16)
    nwin = (en - ast + 1023) >> 10
    def wbody(jw, carry):
      wb = ast + 1024 * jw
      pltpu.sync_copy(y_h.at[pl.ds(_mo8(wb), 1024)], yw)
      pltpu.sync_copy(nmf_h.at[pl.ds(_mo8(wb), 1024)], nmw)
      for d in range(3):
        pltpu.sync_copy(h_h.at[pl.ds(_mo8(d * N_PAD + wb), 1024)],
                        hw.at[pl.ds(d * 1024, 1024)])
      def cbody(cq, c2):
        run, s0, s1, s2 = c2
        sl = pl.ds(16 * cq, 16)
        ai = wb + 16 * cq + iot
        yv = yw[sl]
        nmv = nmw[sl]
        inr = (ai >= st) & (ai < en)
        alive = inr & (nmv > F0)
        rk = _rkey(yv)
        sk = rk ^ I32MIN
        gt = alive & (sk > ts)
        tie = alive & (rk == pfx)
        ti = jnp.where(tie, np.int32(1), np.int32(0))
        csum = plsc.cumsum(ti)
        keep = gt | (tie & ((run + csum) <= rri))
        run = run + jnp.sum(ti)
        th = _tanh(yv)
        keepf = jnp.where(keep, F1, F0)
        hd = hw[pl.ds(16 * cq, 16)] * th
        outw[pl.ds(16 * cq, 16)] = jnp.where(inr, hd, F0)
        s0 = s0 + jnp.sum(jnp.where(keep, hd, F0))
        hd = hw[pl.ds(1024 + 16 * cq, 16)] * th
        outw[pl.ds(1024 + 16 * cq, 16)] = jnp.where(inr, hd, F0)
        s1 = s1 + jnp.sum(jnp.where(keep, hd, F0))
        hd = hw[pl.ds(2048 + 16 * cq, 16)] * th
        outw[pl.ds(2048 + 16 * cq, 16)] = jnp.where(inr, hd, F0)
        s2 = s2 + jnp.sum(jnp.where(keep, hd, F0))
        outw[pl.ds(3072 + 16 * cq, 16)] = jnp.where(inr, keepf, F0)
        return (run, s0, s1, s2)
      c2 = lax.fori_loop(0, 64, cbody, carry)
      for d in range(4):
        _add_rows(outw, d * 1024, s_sh, idxw, d * N_PAD + wb, 1024)
      return c2
    _, s0, s1, s2 = lax.fori_loop(0, nwin, wbody, (np.int32(0), F0, F0, F0))
    kdv = jnp.full((16,), jnp.maximum(k, F1))
    _store1(gapt, 4 * g, jnp.full((16,), s0) / kdv)
    _store1(gapt, 4 * g + 1, jnp.full((16,), s1) / kdv)
    _store1(gapt, 4 * g + 2, jnp.full((16,), s2) / kdv)
    _store1(gapt, 4 * g + 3, k)
  plsc.subcore_barrier()
  _add_rows(gapt, 0, gap_sh, idxw, 0, 256)
  plsc.subcore_barrier()
  for d in range(4):
    pltpu.sync_copy(s_sh.at[pl.ds(_mo8(d * N_PAD + sid * NPT), NPT)], stage)
    @pl.when(cid == 0)
    def _():
      pltpu.sync_copy(stage, ht_o0.at[pl.ds(_mo8(d * N_PAD + sid * NPT), NPT)])
    @pl.when(cid == 1)
    def _():
      pltpu.sync_copy(stage, ht_o1.at[pl.ds(_mo8(d * N_PAD + sid * NPT), NPT)])
  @pl.when(sid == 0)
  def _():
    pltpu.sync_copy(gap_sh, gapt)
    @pl.when(cid == 0)
    def _():
      pltpu.sync_copy(gapt, gap_o0)
    @pl.when(cid == 1)
    def _():
      pltpu.sync_copy(gapt, gap_o1)


@functools.partial(
    pl.kernel,
    out_type=(jax.ShapeDtypeStruct((3 * N_PAD,), jnp.float32),
              jax.ShapeDtypeStruct((N_PAD,), jnp.float32),
              jax.ShapeDtypeStruct((64,), jnp.float32)),
    mesh=MESH,
    compiler_params=pltpu.CompilerParams(needs_layout_passes=False),
    scratch_types=[
        pltpu.VMEM((NPW,), jnp.float32),
        pltpu.VMEM((NPW,), jnp.float32),
        pltpu.VMEM((NPW,), jnp.float32),
        pltpu.VMEM((256,), jnp.float32),
        pltpu.VMEM((256,), jnp.float32),
        pltpu.VMEM((64,), jnp.float32),
    ])
def _k7_comb(ht0_h, ht1_h, gap0_h, gap1_h, h_o, nmf_o, cnt_o,
             a, bq, ob, g0, g1, cb):
  cid = lax.axis_index("c")
  sid = lax.axis_index("s")
  w = cid * NS + sid
  nb = w * NPW
  def plane(dsrc, dst_h, doff):
    pltpu.sync_copy(ht0_h.at[pl.ds(_mo8(dsrc * N_PAD + nb), NPW)], a)
    pltpu.sync_copy(ht1_h.at[pl.ds(_mo8(dsrc * N_PAD + nb), NPW)], bq)
    def body(j, _):
      sl = pl.ds(16 * j, 16)
      ob[sl] = a[sl] + bq[sl]
      return 0
    lax.fori_loop(0, NCHW, body, 0)
    pltpu.sync_copy(ob, dst_h.at[pl.ds(_mo8(doff + nb), NPW)])
  for d in range(3):
    plane(d, h_o, d * N_PAD)
  plane(3, nmf_o, 0)
  @pl.when((cid == 0) & (sid == 0))
  def _():
    pltpu.sync_copy(gap0_h, g0)
    pltpu.sync_copy(gap1_h, g1)
    iot = _iota16()
    def kb(m, _):
      i16 = 16 * m + iot
      kv = (plsc.load_gather(g0, [4 * i16 + 3])
            + plsc.load_gather(g1, [4 * i16 + 3]))
      cb[pl.ds(16 * m, 16)] = kv
      return 0
    lax.fori_loop(0, 4, kb, 0)
    pltpu.sync_copy(cb, cnt_o)


@functools.partial(
    pl.kernel,
    out_type=jax.ShapeDtypeStruct((64,), jnp.float32),
    mesh=MESH,
    compiler_params=pltpu.CompilerParams(needs_layout_passes=False),
    scratch_types=[
        pltpu.VMEM((1536,), jnp.float32),
        pltpu.VMEM((64,), jnp.float32),
        pltpu.VMEM((16,), jnp.float32),
        pltpu.VMEM((16,), jnp.float32),
        pltpu.VMEM((16,), jnp.float32),
        pltpu.VMEM((16,), jnp.float32),
        pltpu.VMEM((16,), jnp.float32),
        pltpu.VMEM((16,), jnp.float32),
    ])
def _k8_head(ga_h, gb_h, gc_h, gd_h, ge_h, gf_h,
             w1_h, b1_h, w2_h, b2_h, w3_h, b3_h, out_h,
             gbuf, ob, w1v, b1v, w2v, b2v, w3v, b3v):
  cid = lax.axis_index("c")
  sid = lax.axis_index("s")
  @pl.when((cid == 0) & (sid == 0))
  def _():
    for i, gh in enumerate((ga_h, gb_h, gc_h, gd_h, ge_h, gf_h)):
      pltpu.sync_copy(gh, gbuf.at[pl.ds(256 * i, 256)])
    pltpu.sync_copy(w1_h, w1v)
    pltpu.sync_copy(b1_h, b1v)
    pltpu.sync_copy(w2_h, w2v)
    pltpu.sync_copy(b2_h, b2v)
    pltpu.sync_copy(w3_h, w3v)
    pltpu.sync_copy(b3_h, b3v)
    w1a = w1v[pl.ds(0, 16)]
    b1a = b1v[pl.ds(0, 16)]
    w2a = w2v[pl.ds(0, 16)]
    b2a = b2v[pl.ds(0, 16)]
    w3a = w3v[pl.ds(0, 16)]
    b3a = b3v[pl.ds(0, 16)]
    iot = _iota16()
    for m in range(4):
      i16 = 16 * m + iot
      z = []
      for d in range(3):
        acc = jnp.zeros((16,), jnp.float32)
        for i in range(6):
          acc = acc + plsc.load_gather(gbuf, [256 * i + 4 * i16 + d])
        z.append(acc)
      a1 = []
      for d in range(3):
        t = z[0] * w1a[d] + z[1] * w1a[3 + d] + z[2] * w1a[6 + d] + b1a[d]
        a1.append(jnp.maximum(t, F0))
      t2 = a1[0] * w2a[0] + a1[1] * w2a[1] + a1[2] * w2a[2] + b2a[0]
      t2 = jnp.maximum(t2, F0)
      t3 = t2 * w3a[0] + b3a[0]
      ob[pl.ds(16 * m, 16)] = F1 / (F1 + jnp.exp(-t3))
    pltpu.sync_copy(ob, out_h)


def kernel(x, edge_index, batch, W1, b1, p1, W2, b2, p2, W3, b3, p3,
           lw1, lb1, lw2, lb2, lw3, lb3):
  f32 = jnp.float32
  row = jnp.concatenate([edge_index[0].astype(jnp.int32),
                         jnp.zeros((E_PAD - E,), jnp.int32)])
  col = jnp.concatenate([edge_index[1].astype(jnp.int32),
                         jnp.full((E_PAD - E,), N, jnp.int32)])
  segp = jnp.concatenate([batch.astype(jnp.int32),
                          jnp.full((N_PAD - N,), 63, jnp.int32)])
  hpl = (jnp.zeros((3, N_PAD), f32).at[:, :N].set(x.astype(f32).T)
         .reshape(3 * N_PAD))
  nmfp = jnp.zeros((N_PAD,), f32).at[:N].set(1.0)
  z1 = jnp.zeros((NPT,), f32)
  z16k = jnp.zeros((HB,), f32)

  def pad16(a):
    return jnp.zeros((16,), f32).at[:a.size].set(a.reshape(-1).astype(f32))

  starts, ends, counts = _k0_segs(segp)
  gaps = []
  for li, (Wl, bl, pvec) in enumerate(((W1, b1, p1), (W2, b2, p2),
                                       (W3, b3, p3))):
    k1 = _k1_deg0 if li == 0 else _k1_deg
    raw0, raw1 = k1(row, col, nmfp, z1)
    u, dinv = _k2_nodeA(raw0, raw1, hpl, nmfp, pad16(Wl))
    s0, s1 = _k3_msg(row, col, u, z1)
    hc, y = _k4_nodeB(s0, s1, u, dinv, pad16(bl), pad16(pvec))
    hist = jnp.zeros((4 * NC * HB,), f32)
    for r in range(4):
      h0, h1 = _k5_hist[r](hist, y, nmfp, segp, counts, z16k)
      hist = (hist.at[(r * NC) * HB:(r * NC + 1) * HB].set(h0)
              .at[(r * NC + 1) * HB:(r * NC + 2) * HB].set(h1))
    ht0, ht1, gap0, gap1 = _k6_keep(hist, starts, ends, counts, y, nmfp,
                                    hc, z16k, z1)
    hpl, nmfp, counts = _k7_comb(ht0, ht1, gap0, gap1)
    gaps.extend([gap0, gap1])
  return _k8_head(gaps[0], gaps[1], gaps[2], gaps[3], gaps[4], gaps[5],
                  pad16(lw1), pad16(lb1), pad16(lw2), pad16(lb2),
                  pad16(lw3), pad16(lb3))


# speedup vs baseline: 2.1596x; 2.1596x over previous
import functools

import jax
import jax.numpy as jnp
import numpy as np
from jax import lax
from jax.experimental import pallas as pl
from jax.experimental.pallas import tpu as pltpu
from jax.experimental.pallas import tpu_sc as plsc

N = 100000
E = 3200000
B = 64
NC, NS, NW = 2, 16, 32
N_PAD = 101376
NPT = N_PAD // NS
NPW = N_PAD // NW
NCHW = NPW // 16
EPW = 100096
E_PAD = EPW * NW
ECH = EPW // 128
HB = B * 256
EBLK = 2176
I32MIN = np.int32(-2147483648)
F1 = np.float32(1.0)
F0 = np.float32(0.0)

MESH = plsc.VectorSubcoreMesh(core_axis_name="c", subcore_axis_name="s")


def _iota16():
  return lax.iota(jnp.int32, 16)


def _mo8(x):
  return pl.multiple_of(x, 8)


def _rsqrt(x):
  i = lax.bitcast_convert_type(x, jnp.int32)
  i = np.int32(0x5F3759DF) - (i >> 1)
  y = lax.bitcast_convert_type(i, jnp.float32)
  for _ in range(3):
    y = y * (np.float32(1.5) - np.float32(0.5) * x * y * y)
  return y


def _tanh(x):
  e = jnp.exp(x + x)
  return F1 - np.float32(2.0) / (e + F1)


def _ceil_pos(t):
  ti = t.astype(jnp.int32).astype(jnp.float32)
  return jnp.where(ti < t, ti + F1, ti)


def _rkey(yv):
  bits = lax.bitcast_convert_type(yv, jnp.int32)
  m = bits >> 31
  return bits ^ (m & np.int32(0x7FFFFFFF)) ^ I32MIN


def _sc32(x):
  return np.int32(x - (1 << 32) if x >= (1 << 31) else x)


def _an_pass(ha, hb, rr):
  def s_body(i, acc):
    v = ha[pl.ds(16 * i, 16)] + hb[pl.ds(16 * i, 16)]
    return acc + jnp.sum(v)
  total = lax.fori_loop(0, 16, s_body, F0)
  thr = total - rr
  def c_body(i, carry):
    run, cnt = carry
    v = ha[pl.ds(16 * i, 16)] + hb[pl.ds(16 * i, 16)]
    cs = plsc.cumsum(v)
    pex = run + cs - v
    cnt = cnt + jnp.sum(jnp.where(pex <= thr, F1, F0))
    return (run + jnp.sum(v), cnt)
  _, cnt = lax.fori_loop(0, 16, c_body, (F0, F0))
  bstar = cnt.astype(jnp.int32) - 1
  def a_body(i, acc):
    v = ha[pl.ds(16 * i, 16)] + hb[pl.ds(16 * i, 16)]
    bidx = 16 * i + _iota16()
    return acc + jnp.sum(jnp.where(bidx <= bstar, v, F0))
  pincl = lax.fori_loop(0, 16, a_body, F0)
  return bstar, rr - (total - pincl)


def _store1(ref, idx, val):
  if getattr(val, "ndim", 0) == 0 or np.ndim(val) == 0:
    val = jnp.full((16,), val)
  plsc.store_scatter(ref, [jnp.full((16,), idx, jnp.int32)],
                     val, mask=_iota16() == 0)


def _sread(ref, idx):
  return plsc.load_gather(ref, [jnp.full((16,), idx, jnp.int32)])[0]


def _add_rows(src_ref, sbase, dst_sh, idxbuf, base, nrow):
  iot = _iota16()
  def body(jj, _):
    for u in range(8):
      idxbuf[pl.ds(16 * u, 16)] = base + 128 * jj + 16 * u + iot
    pltpu.sync_copy(src_ref.at[pl.ds(_mo8(sbase + 128 * jj), 128)],
                    dst_sh.at[idxbuf], add=True)
    return 0
  lax.fori_loop(0, nrow // 128, body, 0)


def _chain(nsteps, hist_h, ha, hb, g, cv):
  c = _sread(cv, g)
  k = _ceil_pos(np.float32(0.8) * c)
  rr = k
  pfx = jnp.int32(0)
  for j in range(nsteps):
    pltpu.sync_copy(hist_h.at[pl.ds(_mo8((j * NC) * HB + g * 256), 256)], ha)
    pltpu.sync_copy(hist_h.at[pl.ds(_mo8((j * NC + 1) * HB + g * 256), 256)], hb)
    bstar, rr = _an_pass(ha, hb, rr)
    pfx = pfx | (bstar << (24 - 8 * j))
  return k, pfx, rr


@functools.partial(
    pl.kernel,
    out_type=(jax.ShapeDtypeStruct((64,), jnp.int32),
              jax.ShapeDtypeStruct((64,), jnp.int32),
              jax.ShapeDtypeStruct((64,), jnp.float32)),
    mesh=MESH,
    compiler_params=pltpu.CompilerParams(needs_layout_passes=False),
    scratch_types=[
        pltpu.VMEM((NPT,), jnp.int32),
        pltpu.VMEM((128,), jnp.float32),
        pltpu.VMEM((64,), jnp.int32),
        pltpu.VMEM((64,), jnp.int32),
        pltpu.VMEM((64,), jnp.float32),
    ])
def _k0_segs(seg_h, st_o, en_o, cn_o, sgw, hist, stv, env, cbv):
  cid = lax.axis_index("c")
  sid = lax.axis_index("s")
  @pl.when((cid == 0) & (sid == 0))
  def _():
    for i in range(8):
      hist[pl.ds(16 * i, 16)] = jnp.zeros((16,), jnp.float32)
    ones = jnp.full((16,), F1)
    def wbody(j, _):
      pltpu.sync_copy(seg_h.at[pl.ds(_mo8(j * NPT), NPT)], sgw)
      def cbody(q, _2):
        gi = j * NPT + 16 * q + _iota16()
        v = sgw[pl.ds(16 * q, 16)]
        plsc.addupdate_scatter(hist, [v], ones, mask=gi < N)
        return 0
      return lax.fori_loop(0, NPT // 16, cbody, 0)
    lax.fori_loop(0, NS, wbody, 0)
    def sbody(m, run):
      v = hist[pl.ds(16 * m, 16)]
      cs = plsc.cumsum(v)
      stv[pl.ds(16 * m, 16)] = (run + cs - v).astype(jnp.int32)
      env[pl.ds(16 * m, 16)] = (run + cs).astype(jnp.int32)
      cbv[pl.ds(16 * m, 16)] = v
      return run + jnp.sum(v)
    lax.fori_loop(0, 4, sbody, F0)
    pltpu.sync_copy(stv, st_o)
    pltpu.sync_copy(env, en_o)
    pltpu.sync_copy(cbv, cn_o)


def _make_deg(masked):
  @functools.partial(
      pl.kernel,
      out_type=(jax.ShapeDtypeStruct((N_PAD,), jnp.float32),
                jax.ShapeDtypeStruct((N_PAD,), jnp.float32)),
      mesh=MESH,
      compiler_params=pltpu.CompilerParams(needs_layout_passes=False),
      scratch_types=[
          pltpu.VMEM((EBLK,), jnp.int32),
          pltpu.VMEM((EBLK,), jnp.int32),
          pltpu.VMEM((128,), jnp.int32),
          pltpu.VMEM((128,), jnp.int32),
          pltpu.VMEM((128,), jnp.float32),
          pltpu.VMEM((NPT,), jnp.float32),
          pltpu.VMEM_SHARED((N_PAD,), jnp.float32),
          pltpu.SemaphoreType.DMA,
      ])
  def k(row_h, col_h, nmf_h, z1_h, o0, o1, colb, rowb, colv, rowv, valv,
        stage, raw_sh, sem):
    cid = lax.axis_index("c")
    sid = lax.axis_index("s")
    w = cid * NS + sid
    pltpu.sync_copy(z1_h, stage)
    pltpu.sync_copy(stage, raw_sh.at[pl.ds(_mo8(sid * NPT), NPT)])
    if not masked:
      for i in range(8):
        valv[pl.ds(16 * i, 16)] = jnp.full((16,), F1)
    plsc.subcore_barrier()
    base = w * EPW
    def body(i, _):
      boff = base + i * EBLK
      dc = pltpu.async_copy(col_h.at[pl.ds(_mo8(boff), EBLK)], colb, sem)
      if masked:
        dr = pltpu.async_copy(row_h.at[pl.ds(_mo8(boff), EBLK)], rowb, sem)
        dr.wait()
      dc.wait()
      def sub(jj, _2):
        for u in range(8):
          su = pl.ds(16 * u, 16)
          lo = pl.ds(128 * jj + 16 * u, 16)
          colv[su] = colb[lo]
          if masked:
            rowv[su] = rowb[lo]
        if masked:
          pltpu.async_copy(nmf_h.at[rowv], valv, sem).wait()
        pltpu.sync_copy(valv, raw_sh.at[colv], add=True)
        return 0
      return lax.fori_loop(0, EBLK // 128, sub, 0)
    lax.fori_loop(0, EPW // EBLK, body, 0)
    plsc.subcore_barrier()
    pltpu.sync_copy(raw_sh.at[pl.ds(_mo8(sid * NPT), NPT)], stage)
    @pl.when(cid == 0)
    def _():
      pltpu.sync_copy(stage, o0.at[pl.ds(_mo8(sid * NPT), NPT)])
    @pl.when(cid == 1)
    def _():
      pltpu.sync_copy(stage, o1.at[pl.ds(_mo8(sid * NPT), NPT)])
  return k

_k1_deg0 = _make_deg(False)
_k1_deg = _make_deg(True)


@functools.partial(
    pl.kernel,
    out_type=(jax.ShapeDtypeStruct((3 * N_PAD,), jnp.float32),
              jax.ShapeDtypeStruct((N_PAD,), jnp.float32)),
    mesh=MESH,
    compiler_params=pltpu.CompilerParams(needs_layout_passes=False),
    scratch_types=[
        pltpu.VMEM((NPW,), jnp.float32),
        pltpu.VMEM((NPW,), jnp.float32),
        pltpu.VMEM((NPW,), jnp.float32),
        pltpu.VMEM((3 * NPW,), jnp.float32),
        pltpu.VMEM((3 * NPW,), jnp.float32),
        pltpu.VMEM((NPW,), jnp.float32),
        pltpu.VMEM((16,), jnp.float32),
    ])
def _k2_nodeA(raw0_h, raw1_h, h_h, nmf_h, wf_h, u_o, dinv_o,
              r0, r1, nmb, hbuf, ubuf, dvb, wv):
  cid = lax.axis_index("c")
  sid = lax.axis_index("s")
  w = cid * NS + sid
  nb = w * NPW
  pltpu.sync_copy(raw0_h.at[pl.ds(_mo8(nb), NPW)], r0)
  pltpu.sync_copy(raw1_h.at[pl.ds(_mo8(nb), NPW)], r1)
  pltpu.sync_copy(nmf_h.at[pl.ds(_mo8(nb), NPW)], nmb)
  for d in range(3):
    pltpu.sync_copy(h_h.at[pl.ds(_mo8(d * N_PAD + nb), NPW)],
                    hbuf.at[pl.ds(d * NPW, NPW)])
  pltpu.sync_copy(wf_h, wv)
  wvec = wv[pl.ds(0, 16)]
  def body(j, _):
    sl = pl.ds(16 * j, 16)
    nmfv = nmb[sl]
    rawv = r0[sl] + r1[sl]
    deg = nmfv * (rawv + F1)
    dv = jnp.where(deg > F0, _rsqrt(deg), F0)
    h0 = hbuf[pl.ds(16 * j, 16)]
    h1 = hbuf[pl.ds(NPW + 16 * j, 16)]
    h2 = hbuf[pl.ds(2 * NPW + 16 * j, 16)]
    for d in range(3):
      xw = h0 * wvec[d] + h1 * wvec[3 + d] + h2 * wvec[6 + d]
      ubuf[pl.ds(d * NPW + 16 * j, 16)] = xw * dv
    dvb[sl] = dv
    return 0
  lax.fori_loop(0, NCHW, body, 0)
  for d in range(3):
    pltpu.sync_copy(ubuf.at[pl.ds(d * NPW, NPW)],
                    u_o.at[pl.ds(_mo8(d * N_PAD + nb), NPW)])
  pltpu.sync_copy(dvb, dinv_o.at[pl.ds(_mo8(nb), NPW)])


@functools.partial(
    pl.kernel,
    out_type=(jax.ShapeDtypeStruct((3 * N_PAD,), jnp.float32),
              jax.ShapeDtypeStruct((3 * N_PAD,), jnp.float32)),
    mesh=MESH,
    compiler_params=pltpu.CompilerParams(needs_layout_passes=False),
    scratch_types=[
        pltpu.VMEM((EBLK,), jnp.int32),
        pltpu.VMEM((EBLK,), jnp.int32),
        pltpu.VMEM((128,), jnp.int32),
        pltpu.VMEM((128,), jnp.int32),
        pltpu.VMEM((128,), jnp.int32),
        pltpu.VMEM((128,), jnp.int32),
        pltpu.VMEM((128,), jnp.int32),
        pltpu.VMEM((128,), jnp.int32),
        pltpu.VMEM((128,), jnp.float32),
        pltpu.VMEM((128,), jnp.float32),
        pltpu.VMEM((128,), jnp.float32),
        pltpu.VMEM((NPT,), jnp.float32),
        pltpu.VMEM_SHARED((3 * N_PAD,), jnp.float32),
        pltpu.SemaphoreType.DMA,
        pltpu.SemaphoreType.DMA,
    ])
def _k3_msg(row_h, col_h, u_h, z1_h, o0, o1,
            colb, rowb, g0, g1, g2, c0, c1, c2, m0, m1, m2,
            stage, s_sh, semg, sems):
  cid = lax.axis_index("c")
  sid = lax.axis_index("s")
  w = cid * NS + sid
  pltpu.sync_copy(z1_h, stage)
  for d in range(3):
    pltpu.sync_copy(stage, s_sh.at[pl.ds(_mo8(d * N_PAD + sid * NPT), NPT)])
  plsc.subcore_barrier()
  base = w * EPW
  def body(i, _):
    boff = base + i * EBLK
    dc = pltpu.async_copy(col_h.at[pl.ds(_mo8(boff), EBLK)], colb, semg)
    dr = pltpu.async_copy(row_h.at[pl.ds(_mo8(boff), EBLK)], rowb, semg)
    dc.wait()
    dr.wait()
    def sub(jj, _2):
      for u in range(8):
        su = pl.ds(16 * u, 16)
        lo = pl.ds(128 * jj + 16 * u, 16)
        rv = rowb[lo]
        cv = colb[lo]
        g0[su] = rv
        g1[su] = rv + np.int32(N_PAD)
        g2[su] = rv + np.int32(2 * N_PAD)
        c0[su] = cv
        c1[su] = cv + np.int32(N_PAD)
        c2[su] = cv + np.int32(2 * N_PAD)
      d0 = pltpu.async_copy(u_h.at[g0], m0, semg)
      d1 = pltpu.async_copy(u_h.at[g1], m1, semg)
      d2 = pltpu.async_copy(u_h.at[g2], m2, semg)
      d0.wait()
      d1.wait()
      d2.wait()
      e0 = pltpu.async_copy(m0, s_sh.at[c0], sems, add=True)
      e1 = pltpu.async_copy(m1, s_sh.at[c1], sems, add=True)
      e2 = pltpu.async_copy(m2, s_sh.at[c2], sems, add=True)
      e0.wait()
      e1.wait()
      e2.wait()
      return 0
    return lax.fori_loop(0, EBLK // 128, sub, 0)
  lax.fori_loop(0, EPW // EBLK, body, 0)
  plsc.subcore_barrier()
  for d in range(3):
    pltpu.sync_copy(s_sh.at[pl.ds(_mo8(d * N_PAD + sid * NPT), NPT)], stage)
    @pl.when(cid == 0)
    def _():
      pltpu.sync_copy(stage, o0.at[pl.ds(_mo8(d * N_PAD + sid * NPT), NPT)])
    @pl.when(cid == 1)
    def _():
      pltpu.sync_copy(stage, o1.at[pl.ds(_mo8(d * N_PAD + sid * NPT), NPT)])


@functools.partial(
    pl.kernel,
    out_type=(jax.ShapeDtypeStruct((3 * N_PAD,), jnp.float32),
              jax.ShapeDtypeStruct((N_PAD,), jnp.float32)),
    mesh=MESH,
    compiler_params=pltpu.CompilerParams(needs_layout_passes=False),
    scratch_types=[
        pltpu.VMEM((3 * NPW,), jnp.float32),
        pltpu.VMEM((3 * NPW,), jnp.float32),
        pltpu.VMEM((3 * NPW,), jnp.float32),
        pltpu.VMEM((NPW,), jnp.float32),
        pltpu.VMEM((3 * NPW,), jnp.float32),
        pltpu.VMEM((NPW,), jnp.float32),
        pltpu.VMEM((16,), jnp.float32),
        pltpu.VMEM((16,), jnp.float32),
    ])
def _k4_nodeB(s0_h, s1_h, u_h, dinv_h, bf_h, pf_h, h_o, y_o,
              s0b, s1b, ub, dvb, hob, yb, bv, pv):
  cid = lax.axis_index("c")
  sid = lax.axis_index("s")
  w = cid * NS + sid
  nb = w * NPW
  for d in range(3):
    pltpu.sync_copy(s0_h.at[pl.ds(_mo8(d * N_PAD + nb), NPW)],
                    s0b.at[pl.ds(d * NPW, NPW)])
    pltpu.sync_copy(s1_h.at[pl.ds(_mo8(d * N_PAD + nb), NPW)],
                    s1b.at[pl.ds(d * NPW, NPW)])
    pltpu.sync_copy(u_h.at[pl.ds(_mo8(d * N_PAD + nb), NPW)],
                    ub.at[pl.ds(d * NPW, NPW)])
  pltpu.sync_copy(dinv_h.at[pl.ds(_mo8(nb), NPW)], dvb)
  pltpu.sync_copy(bf_h, bv)
  pltpu.sync_copy(pf_h, pv)
  pvec = pv[pl.ds(0, 16)]
  bvec = bv[pl.ds(0, 16)]
  pp = jnp.sum(pvec * pvec)
  rsv = _rsqrt(jnp.full((16,), pp))
  def body(j, _):
    sl = pl.ds(16 * j, 16)
    dv = dvb[sl]
    yv = jnp.zeros((16,), jnp.float32)
    for d in range(3):
      dsl = pl.ds(d * NPW + 16 * j, 16)
      sd = s0b[dsl] + s1b[dsl] + ub[dsl]
      hd = jnp.maximum(dv * sd + bvec[d], F0)
      hob[dsl] = hd
      yv = yv + hd * pvec[d]
    yb[sl] = yv * rsv
    return 0
  lax.fori_loop(0, NCHW, body, 0)
  for d in range(3):
    pltpu.sync_copy(hob.at[pl.ds(d * NPW, NPW)],
                    h_o.at[pl.ds(_mo8(d * N_PAD + nb), NPW)])
  pltpu.sync_copy(yb, y_o.at[pl.ds(_mo8(nb), NPW)])


def _make_hist(r):
  maskdec = _sc32((0xFFFFFFFF << (32 - 8 * r)) & 0xFFFFFFFF) if r else np.int32(0)
  sh = 24 - 8 * r

  @functools.partial(
      pl.kernel,
      out_type=(jax.ShapeDtypeStruct((HB,), jnp.float32),
                jax.ShapeDtypeStruct((HB,), jnp.float32)),
      mesh=MESH,
      compiler_params=pltpu.CompilerParams(needs_layout_passes=False),
      scratch_types=[
          pltpu.VMEM((NPW,), jnp.float32),
          pltpu.VMEM((NPW,), jnp.float32),
          pltpu.VMEM((NPW,), jnp.int32),
          pltpu.VMEM((128,), jnp.float32),
          pltpu.VMEM((256,), jnp.float32),
          pltpu.VMEM((256,), jnp.float32),
          pltpu.VMEM((HB,), jnp.float32),
          pltpu.VMEM((128,), jnp.int32),
          pltpu.VMEM((128,), jnp.int32),
          pltpu.VMEM_SHARED((HB,), jnp.float32),
      ])
  def k(hist_h, y_h, nmf_h, seg_h, cnt_h, z16_h, o0, o1,
        yb, nmb, sgb, cv, ha, hb, hp, pfv, idxw, hs_sh):
    cid = lax.axis_index("c")
    sid = lax.axis_index("s")
    w = cid * NS + sid
    nb = w * NPW
    pltpu.sync_copy(y_h.at[pl.ds(_mo8(nb), NPW)], yb)
    pltpu.sync_copy(nmf_h.at[pl.ds(_mo8(nb), NPW)], nmb)
    pltpu.sync_copy(seg_h.at[pl.ds(_mo8(nb), NPW)], sgb)
    pltpu.sync_copy(cnt_h, cv.at[pl.ds(0, 64)])
    pltpu.sync_copy(z16_h, hp)
    @pl.when(sid == 0)
    def _():
      pltpu.sync_copy(hp, hs_sh)
    plsc.subcore_barrier()
    gmin = sgb[pl.ds(0, 16)][0]
    gmax = sgb[pl.ds(NPW - 16, 16)][15]
    def chain_g(g, _):
      _k, pfx, _rr = _chain(r, hist_h, ha, hb, g, cv)
      _store1(pfv, g, pfx)
      return 0
    lax.fori_loop(gmin, gmax + 1, chain_g, 0)
    ones = jnp.full((16,), F1)
    def hbody(j, _):
      sl = pl.ds(16 * j, 16)
      yv = yb[sl]
      nmv = nmb[sl]
      sgv = sgb[sl]
      rk = _rkey(yv)
      pfn = plsc.load_gather(pfv, [sgv])
      match = (nmv > F0) & ((rk & maskdec) == pfn)
      byte = (rk >> sh) & np.int32(255)
      plsc.addupdate_scatter(hp, [sgv * 256 + byte], ones, mask=match)
      return 0
    lax.fori_loop(0, NCHW, hbody, 0)
    plsc.subcore_barrier()
    _add_rows(hp, 0, hs_sh, idxw, 0, HB)
    plsc.subcore_barrier()
    @pl.when(sid == 0)
    def _():
      pltpu.sync_copy(hs_sh, hp)
      @pl.when(cid == 0)
      def _():
        pltpu.sync_copy(hp, o0)
      @pl.when(cid == 1)
      def _():
        pltpu.sync_copy(hp, o1)
  return k

_k5_hist = [_make_hist(r) for r in range(4)]


@functools.partial(
    pl.kernel,
    out_type=(jax.ShapeDtypeStruct((4 * N_PAD,), jnp.float32),
              jax.ShapeDtypeStruct((4 * N_PAD,), jnp.float32),
              jax.ShapeDtypeStruct((256,), jnp.float32),
              jax.ShapeDtypeStruct((256,), jnp.float32)),
    mesh=MESH,
    compiler_params=pltpu.CompilerParams(needs_layout_passes=False),
    scratch_types=[
        pltpu.VMEM((1024,), jnp.float32),
        pltpu.VMEM((1024,), jnp.float32),
        pltpu.VMEM((3 * 1024,), jnp.float32),
        pltpu.VMEM((4 * 1024,), jnp.float32),
        pltpu.VMEM((256,), jnp.float32),
        pltpu.VMEM((256,), jnp.float32),
        pltpu.VMEM((128,), jnp.int32),
        pltpu.VMEM((128,), jnp.int32),
        pltpu.VMEM((128,), jnp.float32),
        pltpu.VMEM((256,), jnp.float32),
        pltpu.VMEM((NPT,), jnp.float32),
        pltpu.VMEM((128,), jnp.int32),
        pltpu.VMEM_SHARED((4 * N_PAD,), jnp.float32),
        pltpu.VMEM_SHARED((256,), jnp.float32),
    ])
def _k6_keep(hist_h, st_h, en_h, cnt_h, y_h, nmf_h, h_h, z16_h, z1_h,
             ht_o0, ht_o1, gap_o0, gap_o1,
             yw, nmw, hw, outw, ha, hb, stv, env, cv, gapt, stage,
             idxw, s_sh, gap_sh):
  cid = lax.axis_index("c")
  sid = lax.axis_index("s")
  w = cid * NS + sid
  pltpu.sync_copy(z1_h, stage)
  for d in range(4):
    pltpu.sync_copy(stage, s_sh.at[pl.ds(_mo8(d * N_PAD + sid * NPT), NPT)])
  pltpu.sync_copy(z16_h.at[pl.ds(0, 256)], gapt)
  @pl.when(sid == 0)
  def _():
    pltpu.sync_copy(gapt, gap_sh)
  pltpu.sync_copy(st_h, stv.at[pl.ds(0, 64)])
  pltpu.sync_copy(en_h, env.at[pl.ds(0, 64)])
  pltpu.sync_copy(cnt_h, cv.at[pl.ds(0, 64)])
  plsc.subcore_barrier()
  iot = _iota16()
  for q in range(2):
    g = 2 * w + q
    k, pfx, rr = _chain(4, hist_h, ha, hb, g, cv)
    ts = pfx ^ I32MIN
    rri = rr.astype(jnp.int32)
    st = _sread(stv, g)
    en = _sread(env, g)
    ast = st & np.int32(-16)
    nwin = (en - ast + 1023) >> 10
    def wbody(jw, carry):
      wb = ast + 1024 * jw
      pltpu.sync_copy(y_h.at[pl.ds(_mo8(wb), 1024)], yw)
      pltpu.sync_copy(nmf_h.at[pl.ds(_mo8(wb), 1024)], nmw)
      for d in range(3):
        pltpu.sync_copy(h_h.at[pl.ds(_mo8(d * N_PAD + wb), 1024)],
                        hw.at[pl.ds(d * 1024, 1024)])
      def cbody(cq, c2):
        run, s0, s1, s2 = c2
        sl = pl.ds(16 * cq, 16)
        ai = wb + 16 * cq + iot
        yv = yw[sl]
        nmv = nmw[sl]
        inr = (ai >= st) & (ai < en)
        alive = inr & (nmv > F0)
        rk = _rkey(yv)
        sk = rk ^ I32MIN
        gt = alive & (sk > ts)
        tie = alive & (rk == pfx)
        ti = jnp.where(tie, np.int32(1), np.int32(0))
        csum = plsc.cumsum(ti)
        keep = gt | (tie & ((run + csum) <= rri))
        run = run + jnp.sum(ti)
        th = _tanh(yv)
        keepf = jnp.where(keep, F1, F0)
        hd = hw[pl.ds(16 * cq, 16)] * th
        outw[pl.ds(16 * cq, 16)] = jnp.where(inr, hd, F0)
        s0 = s0 + jnp.sum(jnp.where(keep, hd, F0))
        hd = hw[pl.ds(1024 + 16 * cq, 16)] * th
        outw[pl.ds(1024 + 16 * cq, 16)] = jnp.where(inr, hd, F0)
        s1 = s1 + jnp.sum(jnp.where(keep, hd, F0))
        hd = hw[pl.ds(2048 + 16 * cq, 16)] * th
        outw[pl.ds(2048 + 16 * cq, 16)] = jnp.where(inr, hd, F0)
        s2 = s2 + jnp.sum(jnp.where(keep, hd, F0))
        outw[pl.ds(3072 + 16 * cq, 16)] = jnp.where(inr, keepf, F0)
        return (run, s0, s1, s2)
      c2 = lax.fori_loop(0, 64, cbody, carry)
      for d in range(4):
        _add_rows(outw, d * 1024, s_sh, idxw, d * N_PAD + wb, 1024)
      return c2
    _, s0, s1, s2 = lax.fori_loop(0, nwin, wbody, (np.int32(0), F0, F0, F0))
    kdv = jnp.full((16,), jnp.maximum(k, F1))
    _store1(gapt, 4 * g, jnp.full((16,), s0) / kdv)
    _store1(gapt, 4 * g + 1, jnp.full((16,), s1) / kdv)
    _store1(gapt, 4 * g + 2, jnp.full((16,), s2) / kdv)
    _store1(gapt, 4 * g + 3, k)
  plsc.subcore_barrier()
  _add_rows(gapt, 0, gap_sh, idxw, 0, 256)
  plsc.subcore_barrier()
  for d in range(4):
    pltpu.sync_copy(s_sh.at[pl.ds(_mo8(d * N_PAD + sid * NPT), NPT)], stage)
    @pl.when(cid == 0)
    def _():
      pltpu.sync_copy(stage, ht_o0.at[pl.ds(_mo8(d * N_PAD + sid * NPT), NPT)])
    @pl.when(cid == 1)
    def _():
      pltpu.sync_copy(stage, ht_o1.at[pl.ds(_mo8(d * N_PAD + sid * NPT), NPT)])
  @pl.when(sid == 0)
  def _():
    pltpu.sync_copy(gap_sh, gapt)
    @pl.when(cid == 0)
    def _():
      pltpu.sync_copy(gapt, gap_o0)
    @pl.when(cid == 1)
    def _():
      pltpu.sync_copy(gapt, gap_o1)


@functools.partial(
    pl.kernel,
    out_type=(jax.ShapeDtypeStruct((3 * N_PAD,), jnp.float32),
              jax.ShapeDtypeStruct((N_PAD,), jnp.float32),
              jax.ShapeDtypeStruct((64,), jnp.float32)),
    mesh=MESH,
    compiler_params=pltpu.CompilerParams(needs_layout_passes=False),
    scratch_types=[
        pltpu.VMEM((NPW,), jnp.float32),
        pltpu.VMEM((NPW,), jnp.float32),
        pltpu.VMEM((NPW,), jnp.float32),
        pltpu.VMEM((256,), jnp.float32),
        pltpu.VMEM((256,), jnp.float32),
        pltpu.VMEM((64,), jnp.float32),
    ])
def _k7_comb(ht0_h, ht1_h, gap0_h, gap1_h, h_o, nmf_o, cnt_o,
             a, bq, ob, g0, g1, cb):
  cid = lax.axis_index("c")
  sid = lax.axis_index("s")
  w = cid * NS + sid
  nb = w * NPW
  def plane(dsrc, dst_h, doff):
    pltpu.sync_copy(ht0_h.at[pl.ds(_mo8(dsrc * N_PAD + nb), NPW)], a)
    pltpu.sync_copy(ht1_h.at[pl.ds(_mo8(dsrc * N_PAD + nb), NPW)], bq)
    def body(j, _):
      sl = pl.ds(16 * j, 16)
      ob[sl] = a[sl] + bq[sl]
      return 0
    lax.fori_loop(0, NCHW, body, 0)
    pltpu.sync_copy(ob, dst_h.at[pl.ds(_mo8(doff + nb), NPW)])
  for d in range(3):
    plane(d, h_o, d * N_PAD)
  plane(3, nmf_o, 0)
  @pl.when((cid == 0) & (sid == 0))
  def _():
    pltpu.sync_copy(gap0_h, g0)
    pltpu.sync_copy(gap1_h, g1)
    iot = _iota16()
    def kb(m, _):
      i16 = 16 * m + iot
      kv = (plsc.load_gather(g0, [4 * i16 + 3])
            + plsc.load_gather(g1, [4 * i16 + 3]))
      cb[pl.ds(16 * m, 16)] = kv
      return 0
    lax.fori_loop(0, 4, kb, 0)
    pltpu.sync_copy(cb, cnt_o)


@functools.partial(
    pl.kernel,
    out_type=jax.ShapeDtypeStruct((64,), jnp.float32),
    mesh=MESH,
    compiler_params=pltpu.CompilerParams(needs_layout_passes=False),
    scratch_types=[
        pltpu.VMEM((1536,), jnp.float32),
        pltpu.VMEM((64,), jnp.float32),
        pltpu.VMEM((16,), jnp.float32),
        pltpu.VMEM((16,), jnp.float32),
        pltpu.VMEM((16,), jnp.float32),
        pltpu.VMEM((16,), jnp.float32),
        pltpu.VMEM((16,), jnp.float32),
        pltpu.VMEM((16,), jnp.float32),
    ])
def _k8_head(ga_h, gb_h, gc_h, gd_h, ge_h, gf_h,
             w1_h, b1_h, w2_h, b2_h, w3_h, b3_h, out_h,
             gbuf, ob, w1v, b1v, w2v, b2v, w3v, b3v):
  cid = lax.axis_index("c")
  sid = lax.axis_index("s")
  @pl.when((cid == 0) & (sid == 0))
  def _():
    for i, gh in enumerate((ga_h, gb_h, gc_h, gd_h, ge_h, gf_h)):
      pltpu.sync_copy(gh, gbuf.at[pl.ds(256 * i, 256)])
    pltpu.sync_copy(w1_h, w1v)
    pltpu.sync_copy(b1_h, b1v)
    pltpu.sync_copy(w2_h, w2v)
    pltpu.sync_copy(b2_h, b2v)
    pltpu.sync_copy(w3_h, w3v)
    pltpu.sync_copy(b3_h, b3v)
    w1a = w1v[pl.ds(0, 16)]
    b1a = b1v[pl.ds(0, 16)]
    w2a = w2v[pl.ds(0, 16)]
    b2a = b2v[pl.ds(0, 16)]
    w3a = w3v[pl.ds(0, 16)]
    b3a = b3v[pl.ds(0, 16)]
    iot = _iota16()
    for m in range(4):
      i16 = 16 * m + iot
      z = []
      for d in range(3):
        acc = jnp.zeros((16,), jnp.float32)
        for i in range(6):
          acc = acc + plsc.load_gather(gbuf, [256 * i + 4 * i16 + d])
        z.append(acc)
      a1 = []
      for d in range(3):
        t = z[0] * w1a[d] + z[1] * w1a[3 + d] + z[2] * w1a[6 + d] + b1a[d]
        a1.append(jnp.maximum(t, F0))
      t2 = a1[0] * w2a[0] + a1[1] * w2a[1] + a1[2] * w2a[2] + b2a[0]
      t2 = jnp.maximum(t2, F0)
      t3 = t2 * w3a[0] + b3a[0]
      ob[pl.ds(16 * m, 16)] = F1 / (F1 + jnp.exp(-t3))
    pltpu.sync_copy(ob, out_h)


def kernel(x, edge_index, batch, W1, b1, p1, W2, b2, p2, W3, b3, p3,
           lw1, lb1, lw2, lb2, lw3, lb3):
  f32 = jnp.float32
  row = jnp.concatenate([edge_index[0].astype(jnp.int32),
                         jnp.zeros((E_PAD - E,), jnp.int32)])
  col = jnp.concatenate([edge_index[1].astype(jnp.int32),
                         jnp.full((E_PAD - E,), N, jnp.int32)])
  segp = jnp.concatenate([batch.astype(jnp.int32),
                          jnp.full((N_PAD - N,), 63, jnp.int32)])
  hpl = (jnp.zeros((3, N_PAD), f32).at[:, :N].set(x.astype(f32).T)
         .reshape(3 * N_PAD))
  nmfp = jnp.zeros((N_PAD,), f32).at[:N].set(1.0)
  z1 = jnp.zeros((NPT,), f32)
  z16k = jnp.zeros((HB,), f32)

  def pad16(a):
    return jnp.zeros((16,), f32).at[:a.size].set(a.reshape(-1).astype(f32))

  starts, ends, counts = _k0_segs(segp)
  gaps = []
  for li, (Wl, bl, pvec) in enumerate(((W1, b1, p1), (W2, b2, p2),
                                       (W3, b3, p3))):
    k1 = _k1_deg0 if li == 0 else _k1_deg
    raw0, raw1 = k1(row, col, nmfp, z1)
    u, dinv = _k2_nodeA(raw0, raw1, hpl, nmfp, pad16(Wl))
    s0, s1 = _k3_msg(row, col, u, z1)
    hc, y = _k4_nodeB(s0, s1, u, dinv, pad16(bl), pad16(pvec))
    hist = jnp.zeros((4 * NC * HB,), f32)
    for r in range(4):
      h0, h1 = _k5_hist[r](hist, y, nmfp, segp, counts, z16k)
      hist = (hist.at[(r * NC) * HB:(r * NC + 1) * HB].set(h0)
              .at[(r * NC + 1) * HB:(r * NC + 2) * HB].set(h1))
    ht0, ht1, gap0, gap1 = _k6_keep(hist, starts, ends, counts, y, nmfp,
                                    hc, z16k, z1)
    hpl, nmfp, counts = _k7_comb(ht0, ht1, gap0, gap1)
    gaps.extend([gap0, gap1])
  return _k8_head(gaps[0], gaps[1], gaps[2], gaps[3], gaps[4], gaps[5],
                  pad16(lw1), pad16(lb1), pad16(lw2), pad16(lb2),
                  pad16(lw3), pad16(lb3))
